# baseline (speedup 1.0000x reference)
import functools

import jax
import jax.numpy as jnp
from jax import lax
from jax.experimental import pallas as pl
from jax.experimental.pallas import tpu as pltpu, tpu_sc as plsc

B = 64
C = 768
H = 14
W = 14
HW = H * W
FREQ = 256
E = 64
K = 8
S = 28
NSTEPS = HW // S

L = 16
NV = E // L

_info = plsc.get_sparse_core_info()
_NC, _NS = _info.num_cores, _info.num_subcores
NW = _NC * _NS
RPW = B // NW


def _pool_body(x_ref, x2_ref, freq_ref, wg_ref, wf_ref, scores_ref, acc_ref):
    g = pl.program_id(0)

    @pl.when(g == 0)
    def _init():
        acc_ref[...] = jnp.zeros_like(acc_ref)

    acc_ref[...] += jnp.sum(x_ref[...], axis=0) + jnp.sum(x2_ref[...], axis=0)

    @pl.when(g == NSTEPS - 1)
    def _epilogue():
        pooled = acc_ref[...] * (1.0 / HW)
        logits = jax.lax.dot_general(
            pooled, wg_ref[...],
            dimension_numbers=(((1,), (1,)), ((), ())),
            preferred_element_type=jnp.float32,
        )
        logits += jax.lax.dot_general(
            freq_ref[...], wf_ref[...],
            dimension_numbers=(((1,), (1,)), ((), ())),
            preferred_element_type=jnp.float32,
        )
        m = jnp.max(logits, axis=-1, keepdims=True)
        ex = jnp.exp(logits - m)
        scores_ref[...] = ex / jnp.sum(ex, axis=-1, keepdims=True)


def _lane_perm(v, perm):
    dnums = lax.GatherDimensionNumbers(
        offset_dims=(), collapsed_slice_dims=(0,), start_index_map=(0,))
    return lax.gather(v, perm[:, None], dnums, slice_sizes=(1,),
                      mode=lax.GatherScatterMode.PROMISE_IN_BOUNDS)


def _bcast_reduce(v, op, lane):
    for sh in (1, 2, 4, 8):
        v = op(v, _lane_perm(v, lane ^ sh))
    return v


_sc_mesh = plsc.VectorSubcoreMesh(core_axis_name="c", subcore_axis_name="s")


@functools.partial(
    pl.kernel, mesh=_sc_mesh,
    out_type=[
        jax.ShapeDtypeStruct((B * E,), jnp.float32),
        jax.ShapeDtypeStruct((B * K,), jnp.int32),
        jax.ShapeDtypeStruct((B * K,), jnp.float32),
    ],
    scratch_types=[
        pltpu.VMEM((E,), jnp.float32),
        pltpu.VMEM((E,), jnp.float32),
        pltpu.VMEM((L,), jnp.int32),
        pltpu.VMEM((L,), jnp.float32),
    ],
)
def _sc_topk(scores_hbm, gates_hbm, idx_hbm, val_hbm,
             scores_v, gates_v, idx_v, val_v):
    wid = lax.axis_index("s") * _NC + lax.axis_index("c")
    lane = lax.iota(jnp.int32, L)
    for r in range(RPW):
        row = wid * RPW + r
        pltpu.sync_copy(scores_hbm.at[pl.ds(row * E, E)], scores_v)
        s = [scores_v[pl.ds(j * L, L)] for j in range(NV)]
        gidx = [lane + j * L for j in range(NV)]
        idx_acc = jnp.zeros((L,), jnp.int32)
        val_acc = jnp.zeros((L,), jnp.float32)
        gv = [jnp.zeros((L,), jnp.float32) for _ in range(NV)]
        neg = jnp.float32(-jnp.inf)
        big = jnp.full((L,), E, jnp.int32)
        for k in range(K):
            m = s[0]
            for j in range(1, NV):
                m = jnp.maximum(m, s[j])
            vmax = _bcast_reduce(m, jnp.maximum, lane)
            cand = big
            for j in range(NV):
                cand = jnp.minimum(cand, jnp.where(s[j] == vmax, gidx[j], big))
            imin = _bcast_reduce(cand, jnp.minimum, lane)
            idx_acc = jnp.where(lane == k, imin, idx_acc)
            val_acc = jnp.where(lane == k, vmax, val_acc)
            for j in range(NV):
                hit = gidx[j] == imin
                gv[j] = jnp.where(hit, vmax, gv[j])
                s[j] = jnp.where(hit, neg, s[j])
        for j in range(NV):
            gates_v[pl.ds(j * L, L)] = gv[j]
        idx_v[...] = idx_acc
        val_v[...] = val_acc
        pltpu.sync_copy(gates_v, gates_hbm.at[pl.ds(row * E, E)])
        pltpu.sync_copy(idx_v.at[pl.ds(0, K)], idx_hbm.at[pl.ds(row * K, K)])
        pltpu.sync_copy(val_v.at[pl.ds(0, K)], val_hbm.at[pl.ds(row * K, K)])


@jax.jit
def kernel(x, freq_emb, W_gate, W_freq):
    xt = jnp.transpose(x, (2, 3, 0, 1)).reshape(HW, B, C)
    scores = pl.pallas_call(
        _pool_body,
        grid=(NSTEPS,),
        in_specs=[
            pl.BlockSpec((S // 2, B, C), lambda g: (2 * g, 0, 0)),
            pl.BlockSpec((S // 2, B, C), lambda g: (2 * g + 1, 0, 0)),
            pl.BlockSpec((B, FREQ), lambda g: (0, 0)),
            pl.BlockSpec((E, C), lambda g: (0, 0)),
            pl.BlockSpec((E, FREQ), lambda g: (0, 0)),
        ],
        out_specs=pl.BlockSpec((B, E), lambda g: (0, 0)),
        out_shape=jax.ShapeDtypeStruct((B, E), jnp.float32),
        scratch_shapes=[pltpu.VMEM((B, C), jnp.float32)],
    )(xt, xt, freq_emb, W_gate, W_freq)
    gates, idx, val = _sc_topk(scores.reshape(B * E))
    return gates.reshape(B, E), idx.reshape(B, K), val.reshape(B, K)


# speedup vs baseline: 2.0459x; 2.0459x over previous
import functools

import jax
import jax.numpy as jnp
from jax import lax
from jax.experimental import pallas as pl
from jax.experimental.pallas import tpu as pltpu, tpu_sc as plsc

B = 64
C = 768
H = 14
W = 14
HW = H * W
FREQ = 256
E = 64
K = 8
S = 28
NSTEPS = HW // S

L = 16
NV = E // L

_info = plsc.get_sparse_core_info()
_NC, _NS = _info.num_cores, _info.num_subcores
NW = _NC * _NS
RPW = B // NW


def _pool_body(x_ref, x2_ref, freq_ref, wg_ref, wf_ref, scores_ref,
               idx_ref, val_ref, acc_ref):
    g = pl.program_id(0)

    @pl.when(g == 0)
    def _init():
        acc_ref[...] = jnp.zeros_like(acc_ref)

    acc_ref[...] += jnp.sum(x_ref[...], axis=0) + jnp.sum(x2_ref[...], axis=0)

    @pl.when(g == NSTEPS - 1)
    def _epilogue():
        pooled = acc_ref[...] * (1.0 / HW)
        logits = jax.lax.dot_general(
            pooled, wg_ref[...],
            dimension_numbers=(((1,), (1,)), ((), ())),
            preferred_element_type=jnp.float32,
        )
        logits += jax.lax.dot_general(
            freq_ref[...], wf_ref[...],
            dimension_numbers=(((1,), (1,)), ((), ())),
            preferred_element_type=jnp.float32,
        )
        m = jnp.max(logits, axis=-1, keepdims=True)
        ex = jnp.exp(logits - m)
        scores = ex / jnp.sum(ex, axis=-1, keepdims=True)
        iota = jax.lax.broadcasted_iota(jnp.int32, (B, E), 1)
        active = jnp.ones((B, E), dtype=jnp.bool_)
        gates = jnp.zeros((B, E), dtype=jnp.float32)
        idxs = []
        vals = []
        for _ in range(K):
            masked = jnp.where(active, scores, -jnp.inf)
            v = jnp.max(masked, axis=-1, keepdims=True)
            cand = jnp.where(masked == v, iota, E)
            i = jnp.min(cand, axis=-1, keepdims=True)
            gates = jnp.where(iota == i, v, gates)
            active = active & (iota != i)
            idxs.append(i)
            vals.append(v)
        scores_ref[...] = gates
        idx_ref[...] = jnp.concatenate(idxs, axis=-1)
        val_ref[...] = jnp.concatenate(vals, axis=-1)


def _lane_perm(v, perm):
    dnums = lax.GatherDimensionNumbers(
        offset_dims=(), collapsed_slice_dims=(0,), start_index_map=(0,))
    return lax.gather(v, perm[:, None], dnums, slice_sizes=(1,),
                      mode=lax.GatherScatterMode.PROMISE_IN_BOUNDS)


def _bcast_reduce(v, op, lane):
    for sh in (1, 2, 4, 8):
        v = op(v, _lane_perm(v, lane ^ sh))
    return v


_sc_mesh = plsc.VectorSubcoreMesh(core_axis_name="c", subcore_axis_name="s")


@functools.partial(
    pl.kernel, mesh=_sc_mesh,
    out_type=[
        jax.ShapeDtypeStruct((B * E,), jnp.float32),
        jax.ShapeDtypeStruct((B * K,), jnp.int32),
        jax.ShapeDtypeStruct((B * K,), jnp.float32),
    ],
    scratch_types=[
        pltpu.VMEM((E,), jnp.float32),
        pltpu.VMEM((E,), jnp.float32),
        pltpu.VMEM((L,), jnp.int32),
        pltpu.VMEM((L,), jnp.float32),
    ],
)
def _sc_topk(scores_hbm, gates_hbm, idx_hbm, val_hbm,
             scores_v, gates_v, idx_v, val_v):
    wid = lax.axis_index("s") * _NC + lax.axis_index("c")
    lane = lax.iota(jnp.int32, L)
    for r in range(RPW):
        row = wid * RPW + r
        pltpu.sync_copy(scores_hbm.at[pl.ds(row * E, E)], scores_v)
        s = [scores_v[pl.ds(j * L, L)] for j in range(NV)]
        gidx = [lane + j * L for j in range(NV)]
        idx_acc = jnp.zeros((L,), jnp.int32)
        val_acc = jnp.zeros((L,), jnp.float32)
        gv = [jnp.zeros((L,), jnp.float32) for _ in range(NV)]
        neg = jnp.float32(-jnp.inf)
        big = jnp.full((L,), E, jnp.int32)
        for k in range(K):
            m = s[0]
            for j in range(1, NV):
                m = jnp.maximum(m, s[j])
            vmax = _bcast_reduce(m, jnp.maximum, lane)
            cand = big
            for j in range(NV):
                cand = jnp.minimum(cand, jnp.where(s[j] == vmax, gidx[j], big))
            imin = _bcast_reduce(cand, jnp.minimum, lane)
            idx_acc = jnp.where(lane == k, imin, idx_acc)
            val_acc = jnp.where(lane == k, vmax, val_acc)
            for j in range(NV):
                hit = gidx[j] == imin
                gv[j] = jnp.where(hit, vmax, gv[j])
                s[j] = jnp.where(hit, neg, s[j])
        for j in range(NV):
            gates_v[pl.ds(j * L, L)] = gv[j]
        idx_v[...] = idx_acc
        val_v[...] = val_acc
        pltpu.sync_copy(gates_v, gates_hbm.at[pl.ds(row * E, E)])
        pltpu.sync_copy(idx_v.at[pl.ds(0, K)], idx_hbm.at[pl.ds(row * K, K)])
        pltpu.sync_copy(val_v.at[pl.ds(0, K)], val_hbm.at[pl.ds(row * K, K)])


@jax.jit
def kernel(x, freq_emb, W_gate, W_freq):
    xt = jnp.transpose(x, (2, 3, 0, 1)).reshape(HW, B, C)
    gates, idx, val = pl.pallas_call(
        _pool_body,
        grid=(NSTEPS,),
        in_specs=[
            pl.BlockSpec((S // 2, B, C), lambda g: (2 * g, 0, 0)),
            pl.BlockSpec((S // 2, B, C), lambda g: (2 * g + 1, 0, 0)),
            pl.BlockSpec((B, FREQ), lambda g: (0, 0)),
            pl.BlockSpec((E, C), lambda g: (0, 0)),
            pl.BlockSpec((E, FREQ), lambda g: (0, 0)),
        ],
        out_specs=[
            pl.BlockSpec((B, E), lambda g: (0, 0)),
            pl.BlockSpec((B, K), lambda g: (0, 0)),
            pl.BlockSpec((B, K), lambda g: (0, 0)),
        ],
        out_shape=[
            jax.ShapeDtypeStruct((B, E), jnp.float32),
            jax.ShapeDtypeStruct((B, K), jnp.int32),
            jax.ShapeDtypeStruct((B, K), jnp.float32),
        ],
        scratch_shapes=[pltpu.VMEM((B, C), jnp.float32)],
    )(xt, xt, freq_emb, W_gate, W_freq)
    return gates, idx, val


# speedup vs baseline: 2.0486x; 1.0013x over previous
import jax
import jax.numpy as jnp
from jax.experimental import pallas as pl
from jax.experimental.pallas import tpu as pltpu

B = 64
C = 768
H = 14
W = 14
HW = H * W
FREQ = 256
E = 64
K = 8
S = 28
NSTEPS = HW // S


def _routing_body(x_ref, x2_ref, freq_ref, wg_ref, wf_ref,
                  gates_ref, idx_ref, val_ref, acc_ref):
    g = pl.program_id(0)

    @pl.when(g == 0)
    def _init():
        acc_ref[...] = jnp.zeros_like(acc_ref)

    acc_ref[...] += jnp.sum(x_ref[...], axis=0) + jnp.sum(x2_ref[...], axis=0)

    @pl.when(g == NSTEPS - 1)
    def _epilogue():
        pooled = acc_ref[...] * (1.0 / HW)
        logits = jax.lax.dot_general(
            pooled, wg_ref[...],
            dimension_numbers=(((1,), (1,)), ((), ())),
            preferred_element_type=jnp.float32,
        )
        logits += jax.lax.dot_general(
            freq_ref[...], wf_ref[...],
            dimension_numbers=(((1,), (1,)), ((), ())),
            preferred_element_type=jnp.float32,
        )

        m = jnp.max(logits, axis=-1, keepdims=True)
        ex = jnp.exp(logits - m)
        scores = ex / jnp.sum(ex, axis=-1, keepdims=True)

        iota = jax.lax.broadcasted_iota(jnp.int32, (B, E), 1)
        active = jnp.ones((B, E), dtype=jnp.bool_)
        gates = jnp.zeros((B, E), dtype=jnp.float32)
        idxs = []
        vals = []
        for _ in range(K):
            masked = jnp.where(active, scores, -jnp.inf)
            v = jnp.max(masked, axis=-1, keepdims=True)
            cand = jnp.where(masked == v, iota, E)
            i = jnp.min(cand, axis=-1, keepdims=True)
            gates = jnp.where(iota == i, v, gates)
            active = active & (iota != i)
            idxs.append(i)
            vals.append(v)

        gates_ref[...] = gates
        idx_ref[...] = jnp.concatenate(idxs, axis=-1)
        val_ref[...] = jnp.concatenate(vals, axis=-1)


@jax.jit
def kernel(x, freq_emb, W_gate, W_freq):
    xt = jnp.transpose(x, (2, 3, 0, 1)).reshape(HW, B, C)
    gates, idx, val = pl.pallas_call(
        _routing_body,
        grid=(NSTEPS,),
        in_specs=[
            pl.BlockSpec((S // 2, B, C), lambda g: (2 * g, 0, 0)),
            pl.BlockSpec((S // 2, B, C), lambda g: (2 * g + 1, 0, 0)),
            pl.BlockSpec((B, FREQ), lambda g: (0, 0)),
            pl.BlockSpec((E, C), lambda g: (0, 0)),
            pl.BlockSpec((E, FREQ), lambda g: (0, 0)),
        ],
        out_specs=[
            pl.BlockSpec((B, E), lambda g: (0, 0)),
            pl.BlockSpec((B, K), lambda g: (0, 0)),
            pl.BlockSpec((B, K), lambda g: (0, 0)),
        ],
        out_shape=[
            jax.ShapeDtypeStruct((B, E), jnp.float32),
            jax.ShapeDtypeStruct((B, K), jnp.int32),
            jax.ShapeDtypeStruct((B, K), jnp.float32),
        ],
        scratch_shapes=[pltpu.VMEM((B, C), jnp.float32)],
    )(xt, xt, freq_emb, W_gate, W_freq)
    return gates, idx, val
